# baseline (speedup 1.0000x reference)
import functools

import jax
import jax.numpy as jnp
from jax import lax
from jax.experimental import pallas as pl
from jax.experimental.pallas import tpu as pltpu
from jax.experimental.pallas import tpu_sc as plsc

B = 4096
L = 50
D = 128

_info = plsc.get_sparse_core_info()
NC, NS = _info.num_cores, _info.num_subcores
NW = NC * NS
N = B * L
PER_W = N // NW
CHUNK = 128
NCHUNK = PER_W // CHUNK


NBUF = 5
NGROUP = NCHUNK // NBUF


def _gather_body(idx_hbm, widx_hbm, table_hbm, out_hbm, idx_v, widx_v, rows_v, gsems, wsems):
    wid = lax.axis_index("s") * NC + lax.axis_index("c")
    pltpu.sync_copy(idx_hbm.at[wid], idx_v)
    pltpu.sync_copy(widx_hbm.at[wid], widx_v)

    def gather_desc(j, b):
        return pltpu.make_async_copy(
            table_hbm.at[idx_v.at[j]], rows_v.at[b], gsems.at[b]
        )

    def write_desc(j, b):
        return pltpu.make_async_copy(
            rows_v.at[b], out_hbm.at[widx_v.at[j]], wsems.at[b]
        )

    LA = NBUF - 2

    for b in range(LA):
        gather_desc(b, b).start()
    for b in range(NBUF):
        gather_desc(b, b).wait()
        write_desc(b, b).start()
        if b >= NBUF - LA:
            write_desc(b - (NBUF - LA), (b - (NBUF - LA)) % NBUF).wait()
        gather_desc(b + LA, (b + LA) % NBUF).start()

    def group(g, carry):
        for b in range(NBUF):
            j = g * NBUF + b
            gather_desc(j, b).wait()
            write_desc(j, b).start()
            write_desc(j - (NBUF - LA), (b - (NBUF - LA)) % NBUF).wait()

            @pl.when(j + LA < NCHUNK)
            def _():
                gather_desc(j + LA, (b + LA) % NBUF).start()

        return carry

    lax.fori_loop(1, NGROUP, group, 0)

    for j in range(NCHUNK - (NBUF - LA), NCHUNK):
        write_desc(j, j % NBUF).wait()


@jax.jit
def _gather(idx, widx, table):
    mesh = plsc.VectorSubcoreMesh(core_axis_name="c", subcore_axis_name="s")
    return pl.kernel(
        _gather_body,
        out_type=jax.ShapeDtypeStruct((N, D), jnp.float32),
        mesh=mesh,
        scratch_types=[
            pltpu.VMEM((NCHUNK, CHUNK), jnp.int32),
            pltpu.VMEM((NCHUNK, CHUNK), jnp.int32),
            pltpu.VMEM((NBUF, CHUNK, D), jnp.float32),
            pltpu.SemaphoreType.DMA((NBUF,)),
            pltpu.SemaphoreType.DMA((NBUF,)),
        ],
    )(idx, widx, table)


def kernel(input_ids, table):
    idx = input_ids.astype(jnp.int32).T.reshape(NW, NCHUNK, CHUNK)
    widx = jnp.arange(N, dtype=jnp.int32).reshape(NW, NCHUNK, CHUNK)
    out = _gather(idx, widx, table)
    return out.reshape(L, B, D).transpose(1, 0, 2)


# speedup vs baseline: 1.0025x; 1.0025x over previous
import functools

import jax
import jax.numpy as jnp
from jax import lax
from jax.experimental import pallas as pl
from jax.experimental.pallas import tpu as pltpu
from jax.experimental.pallas import tpu_sc as plsc

B = 4096
L = 50
D = 128

_info = plsc.get_sparse_core_info()
NC, NS = _info.num_cores, _info.num_subcores
NW = NC * NS
N = B * L
PER_W = N // NW
CHUNK = 128
NCHUNK = PER_W // CHUNK


NBUF = 5
NGROUP = NCHUNK // NBUF


def _gather_body(idx_hbm, widx_hbm, table_hbm, out_hbm, idx_v, widx_v, rows_v, gsems, wsems):
    wid = lax.axis_index("s") * NC + lax.axis_index("c")
    pltpu.sync_copy(idx_hbm.at[wid], idx_v)
    pltpu.sync_copy(widx_hbm.at[wid], widx_v)

    def gather_desc(j, b):
        return pltpu.make_async_copy(
            table_hbm.at[idx_v.at[j]], rows_v.at[b], gsems.at[b]
        )

    def write_desc(j, b):
        if b % 2 == 0:
            dst = out_hbm.at[widx_v.at[j]]
        else:
            dst = out_hbm.at[pl.ds(wid * PER_W + j * CHUNK, CHUNK)]
        return pltpu.make_async_copy(rows_v.at[b], dst, wsems.at[b])

    LA = NBUF - 2

    for b in range(LA):
        gather_desc(b, b).start()
    for b in range(NBUF):
        gather_desc(b, b).wait()
        write_desc(b, b).start()
        if b >= NBUF - LA:
            write_desc(b - (NBUF - LA), (b - (NBUF - LA)) % NBUF).wait()
        gather_desc(b + LA, (b + LA) % NBUF).start()

    def group(g, carry):
        for b in range(NBUF):
            j = g * NBUF + b
            gather_desc(j, b).wait()
            write_desc(j, b).start()
            write_desc(j - (NBUF - LA), (b - (NBUF - LA)) % NBUF).wait()

            @pl.when(j + LA < NCHUNK)
            def _():
                gather_desc(j + LA, (b + LA) % NBUF).start()

        return carry

    lax.fori_loop(1, NGROUP, group, 0)

    for j in range(NCHUNK - (NBUF - LA), NCHUNK):
        write_desc(j, j % NBUF).wait()


@jax.jit
def _gather(idx, widx, table):
    mesh = plsc.VectorSubcoreMesh(core_axis_name="c", subcore_axis_name="s")
    return pl.kernel(
        _gather_body,
        out_type=jax.ShapeDtypeStruct((N, D), jnp.float32),
        mesh=mesh,
        scratch_types=[
            pltpu.VMEM((NCHUNK, CHUNK), jnp.int32),
            pltpu.VMEM((NCHUNK, CHUNK), jnp.int32),
            pltpu.VMEM((NBUF, CHUNK, D), jnp.float32),
            pltpu.SemaphoreType.DMA((NBUF,)),
            pltpu.SemaphoreType.DMA((NBUF,)),
        ],
    )(idx, widx, table)


def kernel(input_ids, table):
    idx = input_ids.astype(jnp.int32).T.reshape(NW, NCHUNK, CHUNK)
    widx = jnp.arange(N, dtype=jnp.int32).reshape(NW, NCHUNK, CHUNK)
    out = _gather(idx, widx, table)
    return out.reshape(L, B, D).transpose(1, 0, 2)


# speedup vs baseline: 1.0185x; 1.0160x over previous
import functools

import jax
import jax.numpy as jnp
from jax import lax
from jax.experimental import pallas as pl
from jax.experimental.pallas import tpu as pltpu
from jax.experimental.pallas import tpu_sc as plsc

B = 4096
L = 50
D = 128

_info = plsc.get_sparse_core_info()
NC, NS = _info.num_cores, _info.num_subcores
NW = NC * NS
N = B * L
PER_W = N // NW
CHUNK = 64
NCHUNK = PER_W // CHUNK


NBUF = 10
NGROUP = NCHUNK // NBUF


def _gather_body(idx_hbm, table_hbm, out_hbm, idx_v, rows_v, gsems, wsems):
    wid = lax.axis_index("s") * NC + lax.axis_index("c")
    base = wid * PER_W
    pltpu.sync_copy(idx_hbm.at[wid], idx_v)

    def gather_desc(j, b):
        return pltpu.make_async_copy(
            table_hbm.at[idx_v.at[j]], rows_v.at[b], gsems.at[b]
        )

    def write_desc(j, b):
        return pltpu.make_async_copy(
            rows_v.at[b], out_hbm.at[pl.ds(base + j * CHUNK, CHUNK)], wsems.at[b]
        )

    LA = NBUF - 3

    for b in range(LA):
        gather_desc(b, b).start()
    for b in range(NBUF):
        gather_desc(b, b).wait()
        write_desc(b, b).start()
        if b >= NBUF - LA:
            write_desc(b - (NBUF - LA), (b - (NBUF - LA)) % NBUF).wait()
        gather_desc(b + LA, (b + LA) % NBUF).start()

    def group(g, carry):
        for b in range(NBUF):
            j = g * NBUF + b
            gather_desc(j, b).wait()
            write_desc(j, b).start()
            write_desc(j - (NBUF - LA), (b - (NBUF - LA)) % NBUF).wait()

            @pl.when(j + LA < NCHUNK)
            def _():
                gather_desc(j + LA, (b + LA) % NBUF).start()

        return carry

    lax.fori_loop(1, NGROUP, group, 0)

    for j in range(NCHUNK - (NBUF - LA), NCHUNK):
        write_desc(j, j % NBUF).wait()


@jax.jit
def _gather(idx, table):
    mesh = plsc.VectorSubcoreMesh(core_axis_name="c", subcore_axis_name="s")
    return pl.kernel(
        _gather_body,
        out_type=jax.ShapeDtypeStruct((N, D), jnp.float32),
        mesh=mesh,
        scratch_types=[
            pltpu.VMEM((NCHUNK, CHUNK), jnp.int32),
            pltpu.VMEM((NBUF, CHUNK, D), jnp.float32),
            pltpu.SemaphoreType.DMA((NBUF,)),
            pltpu.SemaphoreType.DMA((NBUF,)),
        ],
    )(idx, table)


def kernel(input_ids, table):
    idx = input_ids.astype(jnp.int32).T.reshape(NW, NCHUNK, CHUNK)
    out = _gather(idx, table)
    return out.reshape(L, B, D).transpose(1, 0, 2)
